# baseline (speedup 1.0000x reference)
import functools

import jax
import jax.numpy as jnp
from jax import lax
from jax.experimental import pallas as pl
from jax.experimental.pallas import tpu as pltpu
from jax.experimental.pallas import tpu_sc as plsc

D = 768
E = 8
N = 8192
EPAD = 128
BLK = 512
NP = N + E * BLK
NPB = NP // BLK
NC, NS = 2, 16
NW = NC * NS
CPT = N // NW
SUB = 64
NSUB = CPT // SUB

_HI = jax.lax.Precision.HIGHEST


_R, _C = 64, 128
_BT = 512
_NB = N // _BT


def _bdot(a, b):
    return jax.lax.dot_general(
        a.astype(jnp.bfloat16), b.astype(jnp.bfloat16),
        (((1,), (0,)), ((), ())), preferred_element_type=jnp.float32)


def _ab_body(x_ref, wrt_ref, p_ref, q_ref, t128_ref, l64_ref,
             gate_ref, dest_ref, bexp_ref, eid_s):
    i = pl.program_id(0)
    logits = jax.lax.dot_general(
        x_ref[...].astype(jnp.bfloat16), wrt_ref[...].astype(jnp.bfloat16),
        (((1,), (0,)), ((), ())),
        preferred_element_type=jnp.float32)
    lane = jax.lax.broadcasted_iota(jnp.int32, logits.shape, 1)
    logits = jnp.where(lane < E, logits, -1e30)
    m = jnp.max(logits, axis=1, keepdims=True)
    p = jnp.exp(logits - m)
    s = jnp.sum(p, axis=1, keepdims=True)
    gate_ref[...] = jnp.max(p, axis=1, keepdims=True) / s
    is_max = logits == m
    eid = jnp.min(jnp.where(is_max, lane, EPAD), axis=1, keepdims=True)
    eid_s[pl.ds(i * _BT, _BT), :] = eid.astype(jnp.float32)

    @pl.when(i == _NB - 1)
    def _meta():
        v = eid_s[...]
        eidq = v * q_ref[...]
        eid64 = (_bdot(p_ref[...], eidq) + 0.5).astype(jnp.int32)
        t128 = t128_ref[...]
        l64 = l64_ref[...]
        dest_f = jnp.zeros((_R, _C), jnp.float32)
        off = jnp.int32(0)
        off_ends = []
        for e in range(E):
            mask = (eid64 == e).astype(jnp.float32)
            incl = _bdot(mask, t128)
            chunkpre = _bdot(l64, incl)
            rank = chunkpre[:, _C - 1:_C] + incl - mask
            dest_f = dest_f + mask * (off.astype(jnp.float32) + rank)
            cnt = (jnp.sum(mask) + 0.5).astype(jnp.int32)
            nblk = (cnt + BLK - 1) // BLK
            off = off + nblk * BLK
            off_ends.append(off)
        dest_ref[...] = (dest_f + 0.5).astype(jnp.int32)
        bstart = jax.lax.broadcasted_iota(jnp.int32, (NPB, 1), 0) * BLK
        bexp = jnp.zeros((NPB, 1), jnp.int32)
        for e in range(E):
            bexp = bexp + (bstart >= off_ends[e]).astype(jnp.int32)
        bexp_ref[...] = jnp.minimum(bexp, E - 1)


def _router_meta(x, wrt_pad, pmat, qmat, t128, l64):
    return pl.pallas_call(
        _ab_body,
        grid=(_NB,),
        in_specs=[
            pl.BlockSpec((_BT, D), lambda i: (i, 0)),
            pl.BlockSpec((D, EPAD), lambda i: (0, 0)),
            pl.BlockSpec((_R, N), lambda i: (0, 0)),
            pl.BlockSpec((N, _C), lambda i: (0, 0)),
            pl.BlockSpec((_C, _C), lambda i: (0, 0)),
            pl.BlockSpec((_R, _R), lambda i: (0, 0)),
        ],
        out_specs=[
            pl.BlockSpec((_BT, 1), lambda i: (i, 0)),
            pl.BlockSpec((_R, _C), lambda i: (0, 0)),
            pl.BlockSpec((NPB, 1), lambda i: (0, 0)),
        ],
        out_shape=[
            jax.ShapeDtypeStruct((N, 1), jnp.float32),
            jax.ShapeDtypeStruct((_R, _C), jnp.int32),
            jax.ShapeDtypeStruct((NPB, 1), jnp.int32),
        ],
        scratch_shapes=[pltpu.VMEM((N, 1), jnp.float32)],
    )(x, wrt_pad, pmat, qmat, t128, l64)


SUBC = 64
NSUBC = CPT // SUBC


def _dispatch_body(x_hbm, dest_hbm, xs_hbm, r0, r1, i0, i1, s0, s1):
    wid = lax.axis_index("s") * NC + lax.axis_index("c")
    base = wid * CPT
    rows, idx, sem = [r0, r1], [i0, i1], [s0, s1]
    pend = [None, None]
    for s in range(NSUBC):
        b = s & 1
        if pend[b] is not None:
            pend[b].wait()
        off = base + s * SUBC
        pltpu.sync_copy(dest_hbm.at[pl.ds(off, SUBC)], idx[b])
        pltpu.sync_copy(x_hbm.at[pl.ds(off, SUBC)], rows[b])
        pend[b] = pltpu.async_copy(rows[b], xs_hbm.at[idx[b]], sem[b])
    pend[0].wait()
    pend[1].wait()


@functools.cache
def _dispatch():
    return pl.kernel(
        _dispatch_body,
        mesh=plsc.VectorSubcoreMesh(core_axis_name="c", subcore_axis_name="s"),
        out_type=jax.ShapeDtypeStruct((NP, D), jnp.float32),
        scratch_types=[
            pltpu.VMEM((SUBC, D), jnp.float32),
            pltpu.VMEM((SUBC, D), jnp.float32),
            pltpu.VMEM((SUBC,), jnp.int32),
            pltpu.VMEM((SUBC,), jnp.int32),
            pltpu.SemaphoreType.DMA,
            pltpu.SemaphoreType.DMA,
        ],
    )


def _ffn_body(bexp_ref, xs_ref, w_ref, b_ref, ys_ref):
    acc = jax.lax.dot_general(
        xs_ref[...].astype(jnp.bfloat16), w_ref[0].astype(jnp.bfloat16),
        (((1,), (1,)), ((), ())),
        preferred_element_type=jnp.float32)
    ys_ref[...] = acc + b_ref[0]


def _ffn(bexp, xs, we, be_r):
    grid_spec = pltpu.PrefetchScalarGridSpec(
        num_scalar_prefetch=1,
        grid=(NPB,),
        in_specs=[
            pl.BlockSpec((BLK, D), lambda i, bexp: (i, 0)),
            pl.BlockSpec((1, D, D), lambda i, bexp: (bexp[i], 0, 0)),
            pl.BlockSpec((1, 1, D), lambda i, bexp: (bexp[i], 0, 0)),
        ],
        out_specs=pl.BlockSpec((BLK, D), lambda i, bexp: (i, 0)),
    )
    return pl.pallas_call(
        _ffn_body,
        grid_spec=grid_spec,
        out_shape=jax.ShapeDtypeStruct((NP, D), jnp.float32),
    )(bexp, xs, we, be_r)


def _scale_rows(rows_ref, gate_ref):
    def _row(r, _):
        gsplat = plsc.load_gather(gate_ref, [jnp.full((16,), r, jnp.int32)])
        for c in range(D // 16):
            sl = pl.ds(c * 16, 16)
            rows_ref[r, sl] = rows_ref[r, sl] * gsplat
        return 0

    lax.fori_loop(0, SUB, _row, 0)


def _combine_body(ys_hbm, dest_hbm, gate_hbm, out_hbm,
                  r0, r1, i0, i1, g0, g1, sg0, sg1, so0, so1):
    wid = lax.axis_index("s") * NC + lax.axis_index("c")
    base = wid * CPT
    rows, idx, gv = [r0, r1], [i0, i1], [g0, g1]
    sg, so = [sg0, sg1], [so0, so1]
    gpend, opend = [None, None], [None, None]
    for s in (0, 1):
        off = base + s * SUB
        pltpu.sync_copy(dest_hbm.at[pl.ds(off, SUB)], idx[s])
        pltpu.sync_copy(gate_hbm.at[pl.ds(off, SUB)], gv[s])
        gpend[s] = pltpu.async_copy(ys_hbm.at[idx[s]], rows[s], sg[s])
    for s in range(NSUB):
        b = s & 1
        gpend[b].wait()
        _scale_rows(rows[b], gv[b])
        opend[b] = pltpu.async_copy(
            rows[b], out_hbm.at[pl.ds(base + s * SUB, SUB)], so[b])
        if s + 2 < NSUB:
            opend[b].wait()
            off2 = base + (s + 2) * SUB
            pltpu.sync_copy(dest_hbm.at[pl.ds(off2, SUB)], idx[b])
            pltpu.sync_copy(gate_hbm.at[pl.ds(off2, SUB)], gv[b])
            gpend[b] = pltpu.async_copy(ys_hbm.at[idx[b]], rows[b], sg[b])
    opend[0].wait()
    opend[1].wait()


@functools.cache
def _combine():
    return pl.kernel(
        _combine_body,
        mesh=plsc.VectorSubcoreMesh(core_axis_name="c", subcore_axis_name="s"),
        compiler_params=pltpu.CompilerParams(needs_layout_passes=False),
        out_type=jax.ShapeDtypeStruct((N, D), jnp.float32),
        scratch_types=[
            pltpu.VMEM((SUB, D), jnp.float32),
            pltpu.VMEM((SUB, D), jnp.float32),
            pltpu.VMEM((SUB,), jnp.int32),
            pltpu.VMEM((SUB,), jnp.int32),
            pltpu.VMEM((SUB,), jnp.float32),
            pltpu.VMEM((SUB,), jnp.float32),
            pltpu.SemaphoreType.DMA,
            pltpu.SemaphoreType.DMA,
            pltpu.SemaphoreType.DMA,
            pltpu.SemaphoreType.DMA,
        ],
    )


@jax.jit
def kernel(x, Wr, We, be):
    wrt_pad = jnp.zeros((D, EPAD), jnp.float32).at[:, :E].set(Wr.T)
    be_r = be.reshape(E, 1, D)
    t128 = (jax.lax.broadcasted_iota(jnp.int32, (_C, _C), 0)
            <= jax.lax.broadcasted_iota(jnp.int32, (_C, _C), 1)).astype(jnp.float32)
    l64 = (jax.lax.broadcasted_iota(jnp.int32, (_R, _R), 0)
           > jax.lax.broadcasted_iota(jnp.int32, (_R, _R), 1)).astype(jnp.float32)
    pmat = (jax.lax.broadcasted_iota(jnp.int32, (_R, N), 1) // _C
            == jax.lax.broadcasted_iota(jnp.int32, (_R, N), 0)).astype(jnp.float32)
    qmat = (jax.lax.broadcasted_iota(jnp.int32, (N, _C), 0) % _C
            == jax.lax.broadcasted_iota(jnp.int32, (N, _C), 1)).astype(jnp.float32)

    gate, dest64, bexp = _router_meta(x, wrt_pad, pmat, qmat, t128, l64)
    dest = dest64.reshape(N)
    xs = _dispatch()(x, dest)
    ys = _ffn(bexp.reshape(NPB), xs, We, be_r)
    out = _combine()(ys, dest, gate.reshape(N))
    return out


# speedup vs baseline: 1.0081x; 1.0081x over previous
import functools

import jax
import jax.numpy as jnp
from jax import lax
from jax.experimental import pallas as pl
from jax.experimental.pallas import tpu as pltpu
from jax.experimental.pallas import tpu_sc as plsc

D = 768
E = 8
N = 8192
EPAD = 128
BLK = 512
NP = N + E * BLK
NPB = NP // BLK
NC, NS = 2, 16
NW = NC * NS
CPT = N // NW
SUB = 64
NSUB = CPT // SUB

_HI = jax.lax.Precision.HIGHEST


_R, _C = 64, 128
_BT = 512
_NB = N // _BT


def _bdot(a, b):
    return jax.lax.dot_general(
        a.astype(jnp.bfloat16), b.astype(jnp.bfloat16),
        (((1,), (0,)), ((), ())), preferred_element_type=jnp.float32)


def _ab_body(x_ref, wrt_ref, p_ref, q_ref, t128_ref, l64_ref,
             gate_ref, dest_ref, bexp_ref, eid_s):
    i = pl.program_id(0)
    logits = jax.lax.dot_general(
        x_ref[...].astype(jnp.bfloat16), wrt_ref[...].astype(jnp.bfloat16),
        (((1,), (0,)), ((), ())),
        preferred_element_type=jnp.float32)
    lane = jax.lax.broadcasted_iota(jnp.int32, logits.shape, 1)
    logits = jnp.where(lane < E, logits, -1e30)
    m = jnp.max(logits, axis=1, keepdims=True)
    p = jnp.exp(logits - m)
    s = jnp.sum(p, axis=1, keepdims=True)
    gate_ref[...] = jnp.max(p, axis=1, keepdims=True) / s
    is_max = logits == m
    eid = jnp.min(jnp.where(is_max, lane, EPAD), axis=1, keepdims=True)
    eid_s[pl.ds(i * _BT, _BT), :] = eid.astype(jnp.float32)

    @pl.when(i == _NB - 1)
    def _meta():
        v = eid_s[...].astype(jnp.bfloat16)
        eidq = v * q_ref[...]
        eid64 = (_bdot(p_ref[...], eidq) + 0.5).astype(jnp.int32)
        t128 = t128_ref[...]
        l64 = l64_ref[...]
        dest_f = jnp.zeros((_R, _C), jnp.float32)
        off = jnp.int32(0)
        off_ends = []
        for e in range(E):
            mask = (eid64 == e).astype(jnp.float32)
            incl = _bdot(mask, t128)
            chunkpre = _bdot(l64, incl)
            rank = chunkpre[:, _C - 1:_C] + incl - mask
            dest_f = dest_f + mask * (off.astype(jnp.float32) + rank)
            cnt = (jnp.sum(mask) + 0.5).astype(jnp.int32)
            nblk = (cnt + BLK - 1) // BLK
            off = off + nblk * BLK
            off_ends.append(off)
        dest_ref[...] = (dest_f + 0.5).astype(jnp.int32)
        bstart = jax.lax.broadcasted_iota(jnp.int32, (NPB, 1), 0) * BLK
        bexp = jnp.zeros((NPB, 1), jnp.int32)
        for e in range(E):
            bexp = bexp + (bstart >= off_ends[e]).astype(jnp.int32)
        bexp_ref[...] = jnp.minimum(bexp, E - 1)


def _router_meta(x, wrt_pad, pmat, qmat, t128, l64):
    return pl.pallas_call(
        _ab_body,
        grid=(_NB,),
        in_specs=[
            pl.BlockSpec((_BT, D), lambda i: (i, 0)),
            pl.BlockSpec((D, EPAD), lambda i: (0, 0)),
            pl.BlockSpec((_R, N), lambda i: (0, 0)),
            pl.BlockSpec((N, _C), lambda i: (0, 0)),
            pl.BlockSpec((_C, _C), lambda i: (0, 0)),
            pl.BlockSpec((_R, _R), lambda i: (0, 0)),
        ],
        out_specs=[
            pl.BlockSpec((_BT, 1), lambda i: (i, 0)),
            pl.BlockSpec((_R, _C), lambda i: (0, 0)),
            pl.BlockSpec((NPB, 1), lambda i: (0, 0)),
        ],
        out_shape=[
            jax.ShapeDtypeStruct((N, 1), jnp.float32),
            jax.ShapeDtypeStruct((_R, _C), jnp.int32),
            jax.ShapeDtypeStruct((NPB, 1), jnp.int32),
        ],
        scratch_shapes=[pltpu.VMEM((N, 1), jnp.float32)],
    )(x, wrt_pad, pmat, qmat, t128, l64)


SUBC = 64
NSUBC = CPT // SUBC


def _dispatch_body(x_hbm, dest_hbm, xs_hbm, r0, r1, i0, i1, s0, s1):
    wid = lax.axis_index("s") * NC + lax.axis_index("c")
    base = wid * CPT
    rows, idx, sem = [r0, r1], [i0, i1], [s0, s1]
    pend = [None, None]
    for s in range(NSUBC):
        b = s & 1
        if pend[b] is not None:
            pend[b].wait()
        off = base + s * SUBC
        pltpu.sync_copy(dest_hbm.at[pl.ds(off, SUBC)], idx[b])
        pltpu.sync_copy(x_hbm.at[pl.ds(off, SUBC)], rows[b])
        pend[b] = pltpu.async_copy(rows[b], xs_hbm.at[idx[b]], sem[b])
    pend[0].wait()
    pend[1].wait()


@functools.cache
def _dispatch():
    return pl.kernel(
        _dispatch_body,
        mesh=plsc.VectorSubcoreMesh(core_axis_name="c", subcore_axis_name="s"),
        out_type=jax.ShapeDtypeStruct((NP, D), jnp.float32),
        scratch_types=[
            pltpu.VMEM((SUBC, D), jnp.float32),
            pltpu.VMEM((SUBC, D), jnp.float32),
            pltpu.VMEM((SUBC,), jnp.int32),
            pltpu.VMEM((SUBC,), jnp.int32),
            pltpu.SemaphoreType.DMA,
            pltpu.SemaphoreType.DMA,
        ],
    )


def _ffn_body(bexp_ref, xs_ref, w_ref, b_ref, ys_ref):
    acc = jax.lax.dot_general(
        xs_ref[...].astype(jnp.bfloat16), w_ref[0].astype(jnp.bfloat16),
        (((1,), (1,)), ((), ())),
        preferred_element_type=jnp.float32)
    ys_ref[...] = acc + b_ref[0]


def _ffn(bexp, xs, we, be_r):
    grid_spec = pltpu.PrefetchScalarGridSpec(
        num_scalar_prefetch=1,
        grid=(NPB,),
        in_specs=[
            pl.BlockSpec((BLK, D), lambda i, bexp: (i, 0)),
            pl.BlockSpec((1, D, D), lambda i, bexp: (bexp[i], 0, 0)),
            pl.BlockSpec((1, 1, D), lambda i, bexp: (bexp[i], 0, 0)),
        ],
        out_specs=pl.BlockSpec((BLK, D), lambda i, bexp: (i, 0)),
    )
    return pl.pallas_call(
        _ffn_body,
        grid_spec=grid_spec,
        out_shape=jax.ShapeDtypeStruct((NP, D), jnp.float32),
    )(bexp, xs, we, be_r)


def _scale_rows(rows_ref, gate_ref):
    def _row(r, _):
        gsplat = plsc.load_gather(gate_ref, [jnp.full((16,), r, jnp.int32)])
        for c in range(D // 16):
            sl = pl.ds(c * 16, 16)
            rows_ref[r, sl] = rows_ref[r, sl] * gsplat
        return 0

    lax.fori_loop(0, SUB, _row, 0)


def _combine_body(ys_hbm, dest_hbm, gate_hbm, out_hbm,
                  r0, r1, i0, i1, g0, g1, sg0, sg1, so0, so1):
    wid = lax.axis_index("s") * NC + lax.axis_index("c")
    base = wid * CPT
    rows, idx, gv = [r0, r1], [i0, i1], [g0, g1]
    sg, so = [sg0, sg1], [so0, so1]
    gpend, opend = [None, None], [None, None]
    for s in (0, 1):
        off = base + s * SUB
        pltpu.sync_copy(dest_hbm.at[pl.ds(off, SUB)], idx[s])
        pltpu.sync_copy(gate_hbm.at[pl.ds(off, SUB)], gv[s])
        gpend[s] = pltpu.async_copy(ys_hbm.at[idx[s]], rows[s], sg[s])
    for s in range(NSUB):
        b = s & 1
        gpend[b].wait()
        _scale_rows(rows[b], gv[b])
        opend[b] = pltpu.async_copy(
            rows[b], out_hbm.at[pl.ds(base + s * SUB, SUB)], so[b])
        if s + 2 < NSUB:
            opend[b].wait()
            off2 = base + (s + 2) * SUB
            pltpu.sync_copy(dest_hbm.at[pl.ds(off2, SUB)], idx[b])
            pltpu.sync_copy(gate_hbm.at[pl.ds(off2, SUB)], gv[b])
            gpend[b] = pltpu.async_copy(ys_hbm.at[idx[b]], rows[b], sg[b])
    opend[0].wait()
    opend[1].wait()


@functools.cache
def _combine():
    return pl.kernel(
        _combine_body,
        mesh=plsc.VectorSubcoreMesh(core_axis_name="c", subcore_axis_name="s"),
        compiler_params=pltpu.CompilerParams(needs_layout_passes=False),
        out_type=jax.ShapeDtypeStruct((N, D), jnp.float32),
        scratch_types=[
            pltpu.VMEM((SUB, D), jnp.float32),
            pltpu.VMEM((SUB, D), jnp.float32),
            pltpu.VMEM((SUB,), jnp.int32),
            pltpu.VMEM((SUB,), jnp.int32),
            pltpu.VMEM((SUB,), jnp.float32),
            pltpu.VMEM((SUB,), jnp.float32),
            pltpu.SemaphoreType.DMA,
            pltpu.SemaphoreType.DMA,
            pltpu.SemaphoreType.DMA,
            pltpu.SemaphoreType.DMA,
        ],
    )


@jax.jit
def kernel(x, Wr, We, be):
    wrt_pad = jnp.zeros((D, EPAD), jnp.float32).at[:, :E].set(Wr.T)
    be_r = be.reshape(E, 1, D)
    t128 = (jax.lax.broadcasted_iota(jnp.int32, (_C, _C), 0)
            <= jax.lax.broadcasted_iota(jnp.int32, (_C, _C), 1)).astype(jnp.bfloat16)
    l64 = (jax.lax.broadcasted_iota(jnp.int32, (_R, _R), 0)
           > jax.lax.broadcasted_iota(jnp.int32, (_R, _R), 1)).astype(jnp.bfloat16)
    pmat = (jax.lax.broadcasted_iota(jnp.int32, (_R, N), 1) // _C
            == jax.lax.broadcasted_iota(jnp.int32, (_R, N), 0)).astype(jnp.bfloat16)
    qmat = (jax.lax.broadcasted_iota(jnp.int32, (N, _C), 0) % _C
            == jax.lax.broadcasted_iota(jnp.int32, (N, _C), 1)).astype(jnp.bfloat16)

    gate, dest64, bexp = _router_meta(x, wrt_pad, pmat, qmat, t128, l64)
    dest = dest64.reshape(N)
    xs = _dispatch()(x, dest)
    ys = _ffn(bexp.reshape(NPB), xs, We, be_r)
    out = _combine()(ys, dest, gate.reshape(N))
    return out


# speedup vs baseline: 1.0432x; 1.0348x over previous
import functools

import jax
import jax.numpy as jnp
from jax import lax
from jax.experimental import pallas as pl
from jax.experimental.pallas import tpu as pltpu
from jax.experimental.pallas import tpu_sc as plsc

D = 768
E = 8
N = 8192
EPAD = 128
BLK = 512
NP = N + E * BLK
NPB = NP // BLK
NC, NS = 2, 16
NW = NC * NS
CPT = N // NW
SUB = 64
NSUB = CPT // SUB

_HI = jax.lax.Precision.HIGHEST


_R, _C = 64, 128
_BT = 512
_NB = N // _BT


def _bdot(a, b):
    return jax.lax.dot_general(
        a.astype(jnp.bfloat16), b.astype(jnp.bfloat16),
        (((1,), (0,)), ((), ())), preferred_element_type=jnp.float32)


def _router_body(x_ref, wrt_ref, gate_ref, eid_ref):
    logits = jax.lax.dot_general(
        x_ref[...].astype(jnp.bfloat16), wrt_ref[...].astype(jnp.bfloat16),
        (((1,), (0,)), ((), ())),
        preferred_element_type=jnp.float32)
    lane = jax.lax.broadcasted_iota(jnp.int32, logits.shape, 1)
    logits = jnp.where(lane < E, logits, -1e30)
    m = jnp.max(logits, axis=1, keepdims=True)
    p = jnp.exp(logits - m)
    s = jnp.sum(p, axis=1, keepdims=True)
    gate_ref[...] = jnp.max(p, axis=1, keepdims=True) / s
    is_max = logits == m
    eid_ref[...] = jnp.min(jnp.where(is_max, lane, EPAD), axis=1, keepdims=True)


def _router(x, wrt_pad):
    return pl.pallas_call(
        _router_body,
        grid=(_NB,),
        in_specs=[
            pl.BlockSpec((_BT, D), lambda i: (i, 0)),
            pl.BlockSpec((D, EPAD), lambda i: (0, 0)),
        ],
        out_specs=[
            pl.BlockSpec((_BT, 1), lambda i: (i, 0)),
            pl.BlockSpec((_BT, 1), lambda i: (i, 0)),
        ],
        out_shape=[
            jax.ShapeDtypeStruct((N, 1), jnp.float32),
            jax.ShapeDtypeStruct((N, 1), jnp.int32),
        ],
    )(x, wrt_pad)


def _meta_body(eid_ref, t128_ref, l64_ref, dest_ref, bexp_ref):
    eid = eid_ref[...]
    t128 = t128_ref[...]
    l64 = l64_ref[...]
    dest_f = jnp.zeros((_R, _C), jnp.float32)
    off = jnp.int32(0)
    off_ends = []
    for e in range(E):
        mask = (eid == e).astype(jnp.float32)
        incl = _bdot(mask, t128)
        chunkpre = _bdot(l64, incl)
        rank = chunkpre[:, _C - 1:_C] + incl - mask
        dest_f = dest_f + mask * (off.astype(jnp.float32) + rank)
        cnt = (jnp.sum(mask) + 0.5).astype(jnp.int32)
        nblk = (cnt + BLK - 1) // BLK
        off = off + nblk * BLK
        off_ends.append(off)
    dest_ref[...] = (dest_f + 0.5).astype(jnp.int32)
    bstart = jax.lax.broadcasted_iota(jnp.int32, (NPB, 1), 0) * BLK
    bexp = jnp.zeros((NPB, 1), jnp.int32)
    for e in range(E):
        bexp = bexp + (bstart >= off_ends[e]).astype(jnp.int32)
    bexp_ref[...] = jnp.minimum(bexp, E - 1)


def _meta(eid64, t128, l64):
    return pl.pallas_call(
        _meta_body,
        in_specs=[
            pl.BlockSpec((_R, _C), lambda: (0, 0)),
            pl.BlockSpec((_C, _C), lambda: (0, 0)),
            pl.BlockSpec((_R, _R), lambda: (0, 0)),
        ],
        out_specs=[
            pl.BlockSpec((_R, _C), lambda: (0, 0)),
            pl.BlockSpec((NPB, 1), lambda: (0, 0)),
        ],
        out_shape=[
            jax.ShapeDtypeStruct((_R, _C), jnp.int32),
            jax.ShapeDtypeStruct((NPB, 1), jnp.int32),
        ],
    )(eid64, t128, l64)


SUBC = 64
NSUBC = CPT // SUBC


def _dispatch_body(x_hbm, dest_hbm, xs_hbm, r0, r1, i0, i1, s0, s1):
    wid = lax.axis_index("s") * NC + lax.axis_index("c")
    base = wid * CPT
    rows, idx, sem = [r0, r1], [i0, i1], [s0, s1]
    pend = [None, None]
    for s in range(NSUBC):
        b = s & 1
        if pend[b] is not None:
            pend[b].wait()
        off = base + s * SUBC
        pltpu.sync_copy(dest_hbm.at[pl.ds(off, SUBC)], idx[b])
        pltpu.sync_copy(x_hbm.at[pl.ds(off, SUBC)], rows[b])
        pend[b] = pltpu.async_copy(rows[b], xs_hbm.at[idx[b]], sem[b])
    pend[0].wait()
    pend[1].wait()


@functools.cache
def _dispatch():
    return pl.kernel(
        _dispatch_body,
        mesh=plsc.VectorSubcoreMesh(core_axis_name="c", subcore_axis_name="s"),
        out_type=jax.ShapeDtypeStruct((NP, D), jnp.float32),
        scratch_types=[
            pltpu.VMEM((SUBC, D), jnp.float32),
            pltpu.VMEM((SUBC, D), jnp.float32),
            pltpu.VMEM((SUBC,), jnp.int32),
            pltpu.VMEM((SUBC,), jnp.int32),
            pltpu.SemaphoreType.DMA,
            pltpu.SemaphoreType.DMA,
        ],
    )


def _ffn_body(bexp_ref, xs_ref, w_ref, b_ref, ys_ref):
    acc = jax.lax.dot_general(
        xs_ref[...].astype(jnp.bfloat16), w_ref[0].astype(jnp.bfloat16),
        (((1,), (1,)), ((), ())),
        preferred_element_type=jnp.float32)
    ys_ref[...] = acc + b_ref[0]


def _ffn(bexp, xs, we, be_r):
    grid_spec = pltpu.PrefetchScalarGridSpec(
        num_scalar_prefetch=1,
        grid=(NPB,),
        in_specs=[
            pl.BlockSpec((BLK, D), lambda i, bexp: (i, 0)),
            pl.BlockSpec((1, D, D), lambda i, bexp: (bexp[i], 0, 0)),
            pl.BlockSpec((1, 1, D), lambda i, bexp: (bexp[i], 0, 0)),
        ],
        out_specs=pl.BlockSpec((BLK, D), lambda i, bexp: (i, 0)),
    )
    return pl.pallas_call(
        _ffn_body,
        grid_spec=grid_spec,
        out_shape=jax.ShapeDtypeStruct((NP, D), jnp.float32),
    )(bexp, xs, we, be_r)


def _scale_rows(rows_ref, gate_ref):
    def _row(r, _):
        gsplat = plsc.load_gather(gate_ref, [jnp.full((16,), r, jnp.int32)])
        for c in range(D // 16):
            sl = pl.ds(c * 16, 16)
            rows_ref[r, sl] = rows_ref[r, sl] * gsplat
        return 0

    lax.fori_loop(0, SUB, _row, 0)


def _combine_body(ys_hbm, dest_hbm, gate_hbm, out_hbm,
                  r0, r1, i0, i1, g0, g1, sg0, sg1, so0, so1):
    wid = lax.axis_index("s") * NC + lax.axis_index("c")
    base = wid * CPT
    rows, idx, gv = [r0, r1], [i0, i1], [g0, g1]
    sg, so = [sg0, sg1], [so0, so1]
    gpend, opend = [None, None], [None, None]
    for s in (0, 1):
        off = base + s * SUB
        pltpu.sync_copy(dest_hbm.at[pl.ds(off, SUB)], idx[s])
        pltpu.sync_copy(gate_hbm.at[pl.ds(off, SUB)], gv[s])
        gpend[s] = pltpu.async_copy(ys_hbm.at[idx[s]], rows[s], sg[s])
    for s in range(NSUB):
        b = s & 1
        gpend[b].wait()
        _scale_rows(rows[b], gv[b])
        opend[b] = pltpu.async_copy(
            rows[b], out_hbm.at[pl.ds(base + s * SUB, SUB)], so[b])
        if s + 2 < NSUB:
            opend[b].wait()
            off2 = base + (s + 2) * SUB
            pltpu.sync_copy(dest_hbm.at[pl.ds(off2, SUB)], idx[b])
            pltpu.sync_copy(gate_hbm.at[pl.ds(off2, SUB)], gv[b])
            gpend[b] = pltpu.async_copy(ys_hbm.at[idx[b]], rows[b], sg[b])
    opend[0].wait()
    opend[1].wait()


@functools.cache
def _combine():
    return pl.kernel(
        _combine_body,
        mesh=plsc.VectorSubcoreMesh(core_axis_name="c", subcore_axis_name="s"),
        compiler_params=pltpu.CompilerParams(needs_layout_passes=False),
        out_type=jax.ShapeDtypeStruct((N, D), jnp.float32),
        scratch_types=[
            pltpu.VMEM((SUB, D), jnp.float32),
            pltpu.VMEM((SUB, D), jnp.float32),
            pltpu.VMEM((SUB,), jnp.int32),
            pltpu.VMEM((SUB,), jnp.int32),
            pltpu.VMEM((SUB,), jnp.float32),
            pltpu.VMEM((SUB,), jnp.float32),
            pltpu.SemaphoreType.DMA,
            pltpu.SemaphoreType.DMA,
            pltpu.SemaphoreType.DMA,
            pltpu.SemaphoreType.DMA,
        ],
    )


@jax.jit
def kernel(x, Wr, We, be):
    wrt_pad = jnp.zeros((D, EPAD), jnp.float32).at[:, :E].set(Wr.T)
    be_r = be.reshape(E, 1, D)
    t128 = (jax.lax.broadcasted_iota(jnp.int32, (_C, _C), 0)
            <= jax.lax.broadcasted_iota(jnp.int32, (_C, _C), 1)).astype(jnp.bfloat16)
    l64 = (jax.lax.broadcasted_iota(jnp.int32, (_R, _R), 0)
           > jax.lax.broadcasted_iota(jnp.int32, (_R, _R), 1)).astype(jnp.bfloat16)
    gate, eid = _router(x, wrt_pad)
    dest64, bexp = _meta(eid.reshape(_R, _C), t128, l64)
    dest = dest64.reshape(N)
    xs = _dispatch()(x, dest)
    ys = _ffn(bexp.reshape(NPB), xs, We, be_r)
    out = _combine()(ys, dest, gate.reshape(N))
    return out


# speedup vs baseline: 1.0463x; 1.0030x over previous
import functools

import jax
import jax.numpy as jnp
from jax import lax
from jax.experimental import pallas as pl
from jax.experimental.pallas import tpu as pltpu
from jax.experimental.pallas import tpu_sc as plsc

D = 768
E = 8
N = 8192
EPAD = 128
BLK = 512
NP = N + E * BLK
NPB = NP // BLK
NC, NS = 2, 16
NW = NC * NS
CPT = N // NW
SUB = 64
NSUB = CPT // SUB

_HI = jax.lax.Precision.HIGHEST


_R, _C = 64, 128
_BT = 512
_NB = N // _BT


def _bdot(a, b):
    return jax.lax.dot_general(
        a.astype(jnp.bfloat16), b.astype(jnp.bfloat16),
        (((1,), (0,)), ((), ())), preferred_element_type=jnp.float32)


def _router_body(x_ref, wrt_ref, gate_ref, eid_ref):
    logits = jax.lax.dot_general(
        x_ref[...].astype(jnp.bfloat16), wrt_ref[...].astype(jnp.bfloat16),
        (((1,), (0,)), ((), ())),
        preferred_element_type=jnp.float32)
    lane = jax.lax.broadcasted_iota(jnp.int32, logits.shape, 1)
    logits = jnp.where(lane < E, logits, -1e30)
    m = jnp.max(logits, axis=1, keepdims=True)
    p = jnp.exp(logits - m)
    s = jnp.sum(p, axis=1, keepdims=True)
    gate_ref[...] = jnp.max(p, axis=1, keepdims=True) / s
    is_max = logits == m
    eid_ref[...] = jnp.min(jnp.where(is_max, lane, EPAD), axis=1, keepdims=True)


def _router(x, wrt_pad):
    return pl.pallas_call(
        _router_body,
        grid=(_NB,),
        in_specs=[
            pl.BlockSpec((_BT, D), lambda i: (i, 0)),
            pl.BlockSpec((D, EPAD), lambda i: (0, 0)),
        ],
        out_specs=[
            pl.BlockSpec((_BT, 1), lambda i: (i, 0)),
            pl.BlockSpec((_BT, 1), lambda i: (i, 0)),
        ],
        out_shape=[
            jax.ShapeDtypeStruct((N, 1), jnp.float32),
            jax.ShapeDtypeStruct((N, 1), jnp.int32),
        ],
    )(x, wrt_pad)


def _meta_body(eid_ref, t128_ref, l64_ref, dest_ref, bexp_ref):
    eid = eid_ref[...]
    t128 = t128_ref[...]
    l64 = l64_ref[...]
    dest_f = jnp.zeros((_R, _C), jnp.float32)
    off = jnp.int32(0)
    off_ends = []
    for e in range(E):
        mask = (eid == e).astype(jnp.float32)
        incl = _bdot(mask, t128)
        chunkpre = _bdot(l64, incl)
        rank = chunkpre[:, _C - 1:_C] + incl - mask
        dest_f = dest_f + mask * (off.astype(jnp.float32) + rank)
        cnt = (jnp.sum(mask) + 0.5).astype(jnp.int32)
        nblk = (cnt + BLK - 1) // BLK
        off = off + nblk * BLK
        off_ends.append(off)
    dest_ref[...] = (dest_f + 0.5).astype(jnp.int32)
    bidx = jax.lax.broadcasted_iota(jnp.int32, (NPB, 1), 0)
    bstart = bidx * BLK
    bexp = jnp.zeros((NPB, 1), jnp.int32)
    for e in range(E):
        bexp = bexp + (bstart >= off_ends[e]).astype(jnp.int32)
    bexp = jnp.minimum(bexp, E - 1)
    used = bstart < off_ends[E - 1]
    col0 = jnp.where(used, bexp, bexp[0:1, 0:1])
    col1 = jnp.where(used, bidx, 0)
    bexp_ref[...] = jnp.concatenate([col0, col1], axis=1)


def _meta(eid64, t128, l64):
    return pl.pallas_call(
        _meta_body,
        in_specs=[
            pl.BlockSpec((_R, _C), lambda: (0, 0)),
            pl.BlockSpec((_C, _C), lambda: (0, 0)),
            pl.BlockSpec((_R, _R), lambda: (0, 0)),
        ],
        out_specs=[
            pl.BlockSpec((_R, _C), lambda: (0, 0)),
            pl.BlockSpec((NPB, 2), lambda: (0, 0)),
        ],
        out_shape=[
            jax.ShapeDtypeStruct((_R, _C), jnp.int32),
            jax.ShapeDtypeStruct((NPB, 2), jnp.int32),
        ],
    )(eid64, t128, l64)


SUBC = 64
NSUBC = CPT // SUBC


def _dispatch_body(x_hbm, dest_hbm, xs_hbm, r0, r1, i0, i1, s0, s1):
    wid = lax.axis_index("s") * NC + lax.axis_index("c")
    base = wid * CPT
    rows, idx, sem = [r0, r1], [i0, i1], [s0, s1]
    pend = [None, None]
    for s in range(NSUBC):
        b = s & 1
        if pend[b] is not None:
            pend[b].wait()
        off = base + s * SUBC
        pltpu.sync_copy(dest_hbm.at[pl.ds(off, SUBC)], idx[b])
        pltpu.sync_copy(x_hbm.at[pl.ds(off, SUBC)], rows[b])
        pend[b] = pltpu.async_copy(rows[b], xs_hbm.at[idx[b]], sem[b])
    pend[0].wait()
    pend[1].wait()


@functools.cache
def _dispatch():
    return pl.kernel(
        _dispatch_body,
        mesh=plsc.VectorSubcoreMesh(core_axis_name="c", subcore_axis_name="s"),
        out_type=jax.ShapeDtypeStruct((NP, D), jnp.float32),
        scratch_types=[
            pltpu.VMEM((SUBC, D), jnp.float32),
            pltpu.VMEM((SUBC, D), jnp.float32),
            pltpu.VMEM((SUBC,), jnp.int32),
            pltpu.VMEM((SUBC,), jnp.int32),
            pltpu.SemaphoreType.DMA,
            pltpu.SemaphoreType.DMA,
        ],
    )


def _ffn_body(bexp_ref, xs_ref, w_ref, b_ref, ys_ref):
    acc = jax.lax.dot_general(
        xs_ref[...].astype(jnp.bfloat16), w_ref[0].astype(jnp.bfloat16),
        (((1,), (1,)), ((), ())),
        preferred_element_type=jnp.float32)
    ys_ref[...] = acc + b_ref[0]


def _ffn(bexp, xs, we, be_r):
    grid_spec = pltpu.PrefetchScalarGridSpec(
        num_scalar_prefetch=1,
        grid=(NPB,),
        in_specs=[
            pl.BlockSpec((BLK, D), lambda i, bexp: (bexp[i, 1], 0)),
            pl.BlockSpec((1, D, D), lambda i, bexp: (bexp[i, 0], 0, 0)),
            pl.BlockSpec((1, 1, D), lambda i, bexp: (bexp[i, 0], 0, 0)),
        ],
        out_specs=pl.BlockSpec((BLK, D), lambda i, bexp: (bexp[i, 1], 0)),
    )
    return pl.pallas_call(
        _ffn_body,
        grid_spec=grid_spec,
        out_shape=jax.ShapeDtypeStruct((NP, D), jnp.float32),
    )(bexp, xs, we, be_r)


def _scale_rows(rows_ref, gate_ref):
    def _row(r, _):
        gsplat = plsc.load_gather(gate_ref, [jnp.full((16,), r, jnp.int32)])
        for c in range(D // 16):
            sl = pl.ds(c * 16, 16)
            rows_ref[r, sl] = rows_ref[r, sl] * gsplat
        return 0

    lax.fori_loop(0, SUB, _row, 0)


def _combine_body(ys_hbm, dest_hbm, gate_hbm, out_hbm,
                  r0, r1, i0, i1, g0, g1, sg0, sg1, so0, so1):
    wid = lax.axis_index("s") * NC + lax.axis_index("c")
    base = wid * CPT
    rows, idx, gv = [r0, r1], [i0, i1], [g0, g1]
    sg, so = [sg0, sg1], [so0, so1]
    gpend, opend = [None, None], [None, None]
    for s in (0, 1):
        off = base + s * SUB
        pltpu.sync_copy(dest_hbm.at[pl.ds(off, SUB)], idx[s])
        pltpu.sync_copy(gate_hbm.at[pl.ds(off, SUB)], gv[s])
        gpend[s] = pltpu.async_copy(ys_hbm.at[idx[s]], rows[s], sg[s])
    for s in range(NSUB):
        b = s & 1
        gpend[b].wait()
        _scale_rows(rows[b], gv[b])
        opend[b] = pltpu.async_copy(
            rows[b], out_hbm.at[pl.ds(base + s * SUB, SUB)], so[b])
        if s + 2 < NSUB:
            opend[b].wait()
            off2 = base + (s + 2) * SUB
            pltpu.sync_copy(dest_hbm.at[pl.ds(off2, SUB)], idx[b])
            pltpu.sync_copy(gate_hbm.at[pl.ds(off2, SUB)], gv[b])
            gpend[b] = pltpu.async_copy(ys_hbm.at[idx[b]], rows[b], sg[b])
    opend[0].wait()
    opend[1].wait()


@functools.cache
def _combine():
    return pl.kernel(
        _combine_body,
        mesh=plsc.VectorSubcoreMesh(core_axis_name="c", subcore_axis_name="s"),
        compiler_params=pltpu.CompilerParams(needs_layout_passes=False),
        out_type=jax.ShapeDtypeStruct((N, D), jnp.float32),
        scratch_types=[
            pltpu.VMEM((SUB, D), jnp.float32),
            pltpu.VMEM((SUB, D), jnp.float32),
            pltpu.VMEM((SUB,), jnp.int32),
            pltpu.VMEM((SUB,), jnp.int32),
            pltpu.VMEM((SUB,), jnp.float32),
            pltpu.VMEM((SUB,), jnp.float32),
            pltpu.SemaphoreType.DMA,
            pltpu.SemaphoreType.DMA,
            pltpu.SemaphoreType.DMA,
            pltpu.SemaphoreType.DMA,
        ],
    )


@jax.jit
def kernel(x, Wr, We, be):
    wrt_pad = jnp.zeros((D, EPAD), jnp.float32).at[:, :E].set(Wr.T)
    be_r = be.reshape(E, 1, D)
    t128 = (jax.lax.broadcasted_iota(jnp.int32, (_C, _C), 0)
            <= jax.lax.broadcasted_iota(jnp.int32, (_C, _C), 1)).astype(jnp.bfloat16)
    l64 = (jax.lax.broadcasted_iota(jnp.int32, (_R, _R), 0)
           > jax.lax.broadcasted_iota(jnp.int32, (_R, _R), 1)).astype(jnp.bfloat16)
    gate, eid = _router(x, wrt_pad)
    dest64, bexp = _meta(eid.reshape(_R, _C), t128, l64)
    dest = dest64.reshape(N)
    xs = _dispatch()(x, dest)
    ys = _ffn(bexp, xs, We, be_r)
    out = _combine()(ys, dest, gate.reshape(N))
    return out
